# baseline (speedup 1.0000x reference)
import functools

import jax
import jax.numpy as jnp
from jax import lax
from jax.experimental import pallas as pl
from jax.experimental.pallas import tpu as pltpu
from jax.experimental.pallas import tpu_sc as plsc

VOCAB = 1000
V_PAD = 1024
EMBED_DIM = 64
BATCH = 16384

NUM_CORES = 2
NUM_SUBCORES = 16
NUM_WORKERS = NUM_CORES * NUM_SUBCORES
LANES = 16

D_GROUPS = 8
D_PER_G = EMBED_DIM // D_GROUPS
Q_SPLITS = NUM_WORKERS // D_GROUPS
B_PER_Q = BATCH // Q_SPLITS
N_K = B_PER_Q // LANES
L_TILES = V_PAD // 128


@functools.cache
def _emb_kernel():
    mesh = plsc.VectorSubcoreMesh(
        core_axis_name="c", subcore_axis_name="s",
        num_cores=NUM_CORES, num_subcores=NUM_SUBCORES)

    @functools.partial(
        pl.kernel,
        mesh=mesh,
        out_type=jax.ShapeDtypeStruct((EMBED_DIM, BATCH), jnp.float32),
        scratch_types=[
            pltpu.VMEM((L_TILES, D_PER_G, 128), jnp.float32),
            pltpu.VMEM((EMBED_DIM,), jnp.float32),
            pltpu.VMEM((B_PER_Q,), jnp.int32),
            pltpu.VMEM((D_PER_G, B_PER_Q), jnp.float32),
            pltpu.SemaphoreType.DMA,
        ],
        compiler_params=pltpu.CompilerParams(
            use_tc_tiling_on_sc=True, needs_layout_passes=False),
    )
    def body(w_hbm, b_hbm, idx_hbm, out_hbm, w_v, b_v, idx_v, stage_v, sem):
        wid = lax.axis_index("s") * NUM_CORES + lax.axis_index("c")
        g = wid // Q_SPLITS
        q = wid % Q_SPLITS
        pre = [pltpu.async_copy(idx_hbm.at[pl.ds(q * B_PER_Q, B_PER_Q)],
                                idx_v, sem),
               pltpu.async_copy(b_hbm, b_v, sem)]
        for l in range(L_TILES):
            pre.append(pltpu.async_copy(
                w_hbm.at[pl.ds(g * D_PER_G, D_PER_G), pl.ds(l * 128, 128)],
                w_v.at[l], sem))
        for c in pre:
            c.wait()
        biases = [
            plsc.load_gather(b_v, [jnp.full((LANES,), g * D_PER_G + dl,
                                            jnp.int32)])
            for dl in range(D_PER_G)
        ]
        dls = [jnp.full((LANES,), dl, jnp.int32) for dl in range(D_PER_G)]

        @plsc.parallel_loop(0, N_K, step=1, unroll=8)
        def kbody(k):
            col = k * LANES
            i16 = idx_v[pl.ds(col, LANES)]
            l16 = lax.shift_right_logical(i16, 7)
            m16 = lax.bitwise_and(i16, jnp.int32(127))
            for dl in range(D_PER_G):
                v = plsc.load_gather(w_v, [l16, dls[dl], m16])
                stage_v[dl, pl.ds(col, LANES)] = v + biases[dl]
        pltpu.sync_copy(stage_v,
                        out_hbm.at[pl.ds(g * D_PER_G, D_PER_G),
                                   pl.ds(q * B_PER_Q, B_PER_Q)])

    return body


def kernel(x, W, b):
    idx = x.astype(jnp.int32)
    w_pad = jnp.pad(W, ((0, 0), (0, V_PAD - VOCAB)))
    out_t = _emb_kernel()(w_pad, b, idx)
    return out_t.T


# speedup vs baseline: 1.0417x; 1.0417x over previous
import functools

import jax
import jax.numpy as jnp
from jax import lax
from jax.experimental import pallas as pl
from jax.experimental.pallas import tpu as pltpu
from jax.experimental.pallas import tpu_sc as plsc

VOCAB = 1000
V_PAD = 1024
EMBED_DIM = 64
BATCH = 16384

NUM_CORES = 2
NUM_SUBCORES = 16
NUM_WORKERS = NUM_CORES * NUM_SUBCORES
LANES = 16

D_GROUPS = 8
D_PER_G = EMBED_DIM // D_GROUPS
Q_SPLITS = NUM_WORKERS // D_GROUPS
B_PER_Q = BATCH // Q_SPLITS
N_K = B_PER_Q // LANES
L_TILES = V_PAD // 128


@functools.cache
def _emb_kernel():
    mesh = plsc.VectorSubcoreMesh(
        core_axis_name="c", subcore_axis_name="s",
        num_cores=NUM_CORES, num_subcores=NUM_SUBCORES)

    @functools.partial(
        pl.kernel,
        mesh=mesh,
        out_type=jax.ShapeDtypeStruct((EMBED_DIM, BATCH), jnp.float32),
        scratch_types=[
            pltpu.VMEM((L_TILES, D_PER_G, 128), jnp.float32),
            pltpu.VMEM((EMBED_DIM,), jnp.float32),
            pltpu.VMEM((B_PER_Q,), jnp.int32),
            pltpu.VMEM((D_PER_G, B_PER_Q), jnp.float32),
            pltpu.SemaphoreType.DMA,
        ],
        compiler_params=pltpu.CompilerParams(
            use_tc_tiling_on_sc=True, needs_layout_passes=False),
    )
    def body(w_hbm, b_hbm, idx_hbm, out_hbm, w_v, b_v, idx_v, stage_v, sem):
        wid = lax.axis_index("s") * NUM_CORES + lax.axis_index("c")
        g = wid // Q_SPLITS
        q = wid % Q_SPLITS
        pre = [pltpu.async_copy(idx_hbm.at[pl.ds(q * B_PER_Q, B_PER_Q)],
                                idx_v, sem),
               pltpu.async_copy(b_hbm, b_v, sem)]
        for l in range(L_TILES):
            pre.append(pltpu.async_copy(
                w_hbm.at[pl.ds(g * D_PER_G, D_PER_G), pl.ds(l * 128, 128)],
                w_v.at[l], sem))
        for c in pre:
            c.wait()
        biases = [
            plsc.load_gather(b_v, [jnp.full((LANES,), g * D_PER_G + dl,
                                            jnp.int32)])
            for dl in range(D_PER_G)
        ]
        dls = [jnp.full((LANES,), dl, jnp.int32) for dl in range(D_PER_G)]

        @plsc.parallel_loop(0, N_K, step=1, unroll=4)
        def kbody(k):
            col = k * LANES
            i16 = idx_v[pl.ds(col, LANES)]
            l16 = lax.shift_right_logical(i16, 7)
            m16 = lax.bitwise_and(i16, jnp.int32(127))
            for dl in range(D_PER_G):
                v = plsc.load_gather(w_v, [l16, dls[dl], m16])
                stage_v[dl, pl.ds(col, LANES)] = v + biases[dl]
        pltpu.sync_copy(stage_v,
                        out_hbm.at[pl.ds(g * D_PER_G, D_PER_G),
                                   pl.ds(q * B_PER_Q, B_PER_Q)])

    return body


def kernel(x, W, b):
    idx = x.astype(jnp.int32)
    w_pad = jnp.pad(W, ((0, 0), (0, V_PAD - VOCAB)))
    out_t = _emb_kernel()(w_pad, b, idx)
    return out_t.T


# speedup vs baseline: 1.0440x; 1.0022x over previous
import functools

import jax
import jax.numpy as jnp
from jax import lax
from jax.experimental import pallas as pl
from jax.experimental.pallas import tpu as pltpu
from jax.experimental.pallas import tpu_sc as plsc

VOCAB = 1000
V_PAD = 1024
EMBED_DIM = 64
BATCH = 16384

NUM_CORES = 2
NUM_SUBCORES = 16
NUM_WORKERS = NUM_CORES * NUM_SUBCORES
LANES = 16

D_GROUPS = 8
D_PER_G = EMBED_DIM // D_GROUPS
Q_SPLITS = NUM_WORKERS // D_GROUPS
B_PER_Q = BATCH // Q_SPLITS
N_K = B_PER_Q // LANES
L_TILES = V_PAD // 128


@functools.cache
def _emb_kernel():
    mesh = plsc.VectorSubcoreMesh(
        core_axis_name="c", subcore_axis_name="s",
        num_cores=NUM_CORES, num_subcores=NUM_SUBCORES)

    @functools.partial(
        pl.kernel,
        mesh=mesh,
        out_type=jax.ShapeDtypeStruct((EMBED_DIM, BATCH), jnp.float32),
        scratch_types=[
            pltpu.VMEM((D_PER_G, V_PAD), jnp.float32),
            pltpu.VMEM((EMBED_DIM,), jnp.float32),
            pltpu.VMEM((B_PER_Q,), jnp.int32),
            pltpu.VMEM((D_PER_G, B_PER_Q), jnp.float32),
            pltpu.SemaphoreType.DMA,
        ],
        compiler_params=pltpu.CompilerParams(
            use_tc_tiling_on_sc=True, needs_layout_passes=False),
    )
    def body(w_hbm, b_hbm, idx_hbm, out_hbm, w_v, b_v, idx_v, stage_v, sem):
        wid = lax.axis_index("s") * NUM_CORES + lax.axis_index("c")
        g = wid // Q_SPLITS
        q = wid % Q_SPLITS
        pre = [pltpu.async_copy(idx_hbm.at[pl.ds(q * B_PER_Q, B_PER_Q)],
                                idx_v, sem),
               pltpu.async_copy(b_hbm, b_v, sem)]
        pre.append(pltpu.async_copy(
            w_hbm.at[pl.ds(g * D_PER_G, D_PER_G)], w_v, sem))
        for c in pre:
            c.wait()
        biases = [
            plsc.load_gather(b_v, [jnp.full((LANES,), g * D_PER_G + dl,
                                            jnp.int32)])
            for dl in range(D_PER_G)
        ]
        dls = [jnp.full((LANES,), dl, jnp.int32) for dl in range(D_PER_G)]

        @plsc.parallel_loop(0, N_K, step=1, unroll=4)
        def kbody(k):
            col = k * LANES
            i16 = idx_v[pl.ds(col, LANES)]
            for dl in range(D_PER_G):
                v = plsc.load_gather(w_v, [dls[dl], i16])
                stage_v[dl, pl.ds(col, LANES)] = v + biases[dl]
        pltpu.sync_copy(stage_v,
                        out_hbm.at[pl.ds(g * D_PER_G, D_PER_G),
                                   pl.ds(q * B_PER_Q, B_PER_Q)])

    return body


def kernel(x, W, b):
    idx = x.astype(jnp.int32)
    w_pad = jnp.pad(W, ((0, 0), (0, V_PAD - VOCAB)))
    out_t = _emb_kernel()(w_pad, b, idx)
    return out_t.T


# speedup vs baseline: 1.0562x; 1.0117x over previous
import functools

import jax
import jax.numpy as jnp
from jax import lax
from jax.experimental import pallas as pl
from jax.experimental.pallas import tpu as pltpu
from jax.experimental.pallas import tpu_sc as plsc

VOCAB = 1000
V_PAD = 1024
EMBED_DIM = 64
BATCH = 16384

NUM_CORES = 2
NUM_SUBCORES = 16
NUM_WORKERS = NUM_CORES * NUM_SUBCORES
LANES = 16

D_GROUPS = 8
D_PER_G = EMBED_DIM // D_GROUPS
Q_SPLITS = NUM_WORKERS // D_GROUPS
B_PER_Q = BATCH // Q_SPLITS
N_K = B_PER_Q // LANES
L_TILES = V_PAD // 128


@functools.cache
def _emb_kernel():
    mesh = plsc.VectorSubcoreMesh(
        core_axis_name="c", subcore_axis_name="s",
        num_cores=NUM_CORES, num_subcores=NUM_SUBCORES)

    @functools.partial(
        pl.kernel,
        mesh=mesh,
        out_type=jax.ShapeDtypeStruct((EMBED_DIM, BATCH), jnp.float32),
        scratch_types=[
            pltpu.VMEM((D_PER_G, VOCAB), jnp.float32),
            pltpu.VMEM((EMBED_DIM,), jnp.float32),
            pltpu.VMEM((B_PER_Q,), jnp.int32),
            pltpu.VMEM((D_PER_G, B_PER_Q), jnp.float32),
            pltpu.SemaphoreType.DMA,
        ],
        compiler_params=pltpu.CompilerParams(
            use_tc_tiling_on_sc=True, needs_layout_passes=False),
    )
    def body(w_hbm, b_hbm, idx_hbm, out_hbm, w_v, b_v, idx_v, stage_v, sem):
        wid = lax.axis_index("s") * NUM_CORES + lax.axis_index("c")
        g = wid // Q_SPLITS
        q = wid % Q_SPLITS
        pre = [pltpu.async_copy(idx_hbm.at[pl.ds(q * B_PER_Q, B_PER_Q)],
                                idx_v, sem),
               pltpu.async_copy(b_hbm, b_v, sem)]
        pre.append(pltpu.async_copy(
            w_hbm.at[pl.ds(g * D_PER_G, D_PER_G)], w_v, sem))
        for c in pre:
            c.wait()
        biases = [
            plsc.load_gather(b_v, [jnp.full((LANES,), g * D_PER_G + dl,
                                            jnp.int32)])
            for dl in range(D_PER_G)
        ]
        dls = [jnp.full((LANES,), dl, jnp.int32) for dl in range(D_PER_G)]

        @plsc.parallel_loop(0, N_K, step=1, unroll=4)
        def kbody(k):
            col = k * LANES
            i16 = idx_v[pl.ds(col, LANES)]
            for dl in range(D_PER_G):
                v = plsc.load_gather(w_v, [dls[dl], i16])
                stage_v[dl, pl.ds(col, LANES)] = v + biases[dl]
        pltpu.sync_copy(stage_v,
                        out_hbm.at[pl.ds(g * D_PER_G, D_PER_G),
                                   pl.ds(q * B_PER_Q, B_PER_Q)])

    return body


def kernel(x, W, b):
    idx = x.astype(jnp.int32)
    out_t = _emb_kernel()(W, b, idx)
    return out_t.T
